# baseline (speedup 1.0000x reference)
import functools

import jax
import jax.numpy as jnp
from jax import lax
from jax.experimental import pallas as pl
from jax.experimental.pallas import tpu as pltpu
from jax.experimental.pallas import tpu_sc as plsc

N = 10000
NP = 10240
D = 128
G = 64
E = 320000
NC = 2
NS = 16
NW = NC * NS
CH = 128
K0 = 47
K1 = 111
SRCB_W = max(K0, K1) * CH
EP = NS * (K0 + K1) * CH
ROWS_PER_TILE = NP // NS
BLK = 256
NBLK = NP // BLK
NEG = -3.4e38


def _sc_mesh():
    return plsc.VectorSubcoreMesh(
        core_axis_name="c", subcore_axis_name="s", num_cores=NC)


@functools.lru_cache(maxsize=None)
def _build_sc_agg(with_deg):
    out_type = [jax.ShapeDtypeStruct((2 * NP, D), jnp.float32)]
    scratch = [
        pltpu.VMEM((SRCB_W,), jnp.int32),
        pltpu.VMEM((CH,), jnp.int32),
        pltpu.VMEM((CH,), jnp.int32),
        pltpu.VMEM((CH, D), jnp.float32),
        pltpu.VMEM((CH, D), jnp.float32),
        pltpu.VMEM_SHARED((NP, D), jnp.float32),
        pltpu.SemaphoreType.DMA,
        pltpu.SemaphoreType.DMA,
        pltpu.SemaphoreType.DMA,
        pltpu.SemaphoreType.DMA,
        pltpu.SemaphoreType.DMA,
        pltpu.SemaphoreType.DMA,
    ]
    if with_deg:
        out_type.append(jax.ShapeDtypeStruct((2 * NP,), jnp.float32))
        scratch += [
            pltpu.VMEM((CH,), jnp.float32),
            pltpu.VMEM_SHARED((NP,), jnp.float32),
            pltpu.SemaphoreType.DMA,
        ]
    body = _sc_agg_deg_body if with_deg else _sc_agg_body
    return functools.partial(
        pl.kernel,
        mesh=_sc_mesh(),
        out_type=tuple(out_type),
        scratch_types=tuple(scratch),
    )(body)


def _sc_agg_deg(hw, src3, dst3, z2d, z1d):
    part, degpart = _build_sc_agg(True)(hw, src3, dst3, z2d, z1d)
    return part, degpart


def _sc_agg(hw, src3, dst3, z2d):
    out = _build_sc_agg(False)(hw, src3, dst3, z2d)
    return out[0] if isinstance(out, (tuple, list)) else out


def _sc_pipeline(nchunk, hw, acc, srcb, dstw, dbase, rows0, rows1,
                 dstv0, dstv1, gs0, gs1, ds0, ds1, deg_issue):
    def srcb_row(j):
        return srcb.at[pl.ds(pl.multiple_of(j * CH, 8), CH)]

    def dst_row(j):
        return dstw.at[pl.ds(pl.multiple_of(dbase + j * CH, 8), CH)]

    def issue(j, rbuf, gsem, dbuf, dsem):
        pltpu.async_copy(hw.at[srcb_row(j)], rbuf, gsem)
        pltpu.async_copy(dst_row(j), dbuf, dsem)

    def wait_in(j, rbuf, gsem, dbuf, dsem):
        pltpu.make_async_copy(hw.at[srcb_row(j)], rbuf, gsem).wait()
        pltpu.make_async_copy(dst_row(j), dbuf, dsem).wait()

    def scatter(rbuf, dbuf):
        pltpu.sync_copy(rbuf, acc.at[dbuf], add=True)
        deg_issue(dbuf)

    issue(0, rows0, gs0, dstv0, ds0)

    def body(i, carry):
        j0 = i * 2
        issue(j0 + 1, rows1, gs1, dstv1, ds1)
        wait_in(j0, rows0, gs0, dstv0, ds0)
        scatter(rows0, dstv0)
        issue(j0 + 2, rows0, gs0, dstv0, ds0)
        wait_in(j0 + 1, rows1, gs1, dstv1, ds1)
        scatter(rows1, dstv1)
        return carry

    lax.fori_loop(0, (nchunk - 1) // 2, body, 0)
    assert nchunk % 2 == 1
    wait_in(nchunk - 1, rows0, gs0, dstv0, ds0)
    scatter(rows0, dstv0)


def _sc_common(hw, src3, dst3, z2d, part, srcb, dstv0, dstv1, rows0, rows1,
               acc, gs0, gs1, ds0, ds1, ss0, ss1, deg_issue, deg_drain):
    c = lax.axis_index("c")
    s = lax.axis_index("s")
    pltpu.sync_copy(z2d, acc.at[pl.ds(s * ROWS_PER_TILE, ROWS_PER_TILE)])
    plsc.subcore_barrier()
    base0 = pl.multiple_of(s * (K0 * CH), 8)
    base1 = pl.multiple_of(NS * (K0 * CH) + s * (K1 * CH), 8)

    @pl.when(c == 0)
    def _():
        pltpu.sync_copy(src3.at[pl.ds(base0, K0 * CH)],
                        srcb.at[pl.ds(0, K0 * CH)])
        _sc_pipeline(K0, hw, acc, srcb, dst3, base0, rows0, rows1,
                     dstv0, dstv1, gs0, gs1, ds0, ds1, deg_issue)

    @pl.when(c == 1)
    def _():
        pltpu.sync_copy(src3.at[pl.ds(base1, K1 * CH)],
                        srcb.at[pl.ds(0, K1 * CH)])
        _sc_pipeline(K1, hw, acc, srcb, dst3, base1, rows0, rows1,
                     dstv0, dstv1, gs0, gs1, ds0, ds1, deg_issue)

    plsc.subcore_barrier()
    ro = pl.multiple_of(s * ROWS_PER_TILE, 8)
    oo = pl.multiple_of(c * NP + s * ROWS_PER_TILE, 8)
    pltpu.sync_copy(acc.at[pl.ds(ro, ROWS_PER_TILE)],
                    part.at[pl.ds(oo, ROWS_PER_TILE)])
    return c, s, ro, oo


def _sc_agg_deg_body(hw, src3, dst3, z2d, z1d, part, degpart,
                     srcb, dstv0, dstv1, rows0, rows1, acc,
                     gs0, gs1, ds0, ds1, ss0, ss1, ones_v, dacc, dg):
    s = lax.axis_index("s")
    pltpu.sync_copy(z1d, dacc.at[pl.ds(s * ROWS_PER_TILE, ROWS_PER_TILE)])
    for t in range(CH // 16):
        ones_v[pl.ds(t * 16, 16)] = jnp.full((16,), 1.0, jnp.float32)

    def deg_issue(dbuf):
        pltpu.sync_copy(ones_v, dacc.at[dbuf], add=True)

    def deg_drain():
        pass

    _, _, ro, oo = _sc_common(hw, src3, dst3, z2d, part, srcb, dstv0, dstv1,
                              rows0, rows1, acc, gs0, gs1, ds0, ds1,
                              ss0, ss1, deg_issue, deg_drain)
    pltpu.sync_copy(dacc.at[pl.ds(ro, ROWS_PER_TILE)],
                    degpart.at[pl.ds(oo, ROWS_PER_TILE)])


def _sc_agg_body(hw, src3, dst3, z2d, part, srcb, dstv0, dstv1, rows0, rows1,
                 acc, gs0, gs1, ds0, ds1, ss0, ss1):
    _sc_common(hw, src3, dst3, z2d, part, srcb, dstv0, dstv1, rows0, rows1,
               acc, gs0, gs1, ds0, ds1, ss0, ss1,
               lambda dbuf: None, lambda: None)


def _dot(a, b):
    return jnp.dot(a, b, preferred_element_type=jnp.float32)


def _dotx(a, b):
    return jnp.dot(a, b, preferred_element_type=jnp.float32,
                   precision=lax.Precision.HIGHEST)


def _mm_bias_body(x_ref, w_ref, b_ref, o_ref):
    o_ref[...] = (
        _dot(x_ref[...], w_ref[...])
        + b_ref[...]
    )


def _tc_mm_bias(x, w, b):
    return pl.pallas_call(
        _mm_bias_body,
        grid=(NBLK,),
        in_specs=[
            pl.BlockSpec((BLK, D), lambda i: (i, 0)),
            pl.BlockSpec((D, D), lambda i: (0, 0)),
            pl.BlockSpec((1, D), lambda i: (0, 0)),
        ],
        out_specs=pl.BlockSpec((BLK, D), lambda i: (i, 0)),
        out_shape=jax.ShapeDtypeStruct((NP, D), jnp.float32),
    )(x, w, b)


def _layer2_body(p0, p1, d0, d1, hw, w_ref, b_ref, o_hw, o_deg):
    deg = jnp.maximum(d0[...] + d1[...], 1.0)
    h = jax.nn.relu((p0[...] + p1[...]) / deg + hw[...])
    o_hw[...] = _dot(h, w_ref[...]) + b_ref[...]
    o_deg[...] = deg


def _tc_layer2(p0, p1, d0, d1, hw, w, b):
    return pl.pallas_call(
        _layer2_body,
        grid=(NBLK,),
        in_specs=[
            pl.BlockSpec((BLK, D), lambda i: (i, 0)),
            pl.BlockSpec((BLK, D), lambda i: (i, 0)),
            pl.BlockSpec((BLK, 1), lambda i: (i, 0)),
            pl.BlockSpec((BLK, 1), lambda i: (i, 0)),
            pl.BlockSpec((BLK, D), lambda i: (i, 0)),
            pl.BlockSpec((D, D), lambda i: (0, 0)),
            pl.BlockSpec((1, D), lambda i: (0, 0)),
        ],
        out_specs=[
            pl.BlockSpec((BLK, D), lambda i: (i, 0)),
            pl.BlockSpec((BLK, 1), lambda i: (i, 0)),
        ],
        out_shape=[
            jax.ShapeDtypeStruct((NP, D), jnp.float32),
            jax.ShapeDtypeStruct((NP, 1), jnp.float32),
        ],
    )(p0, p1, d0, d1, hw, w, b)


def _layer3_body(p0, p1, deg, hw, w_ref, b_ref, o_hw):
    h = jax.nn.relu((p0[...] + p1[...]) / deg[...] + hw[...])
    o_hw[...] = _dot(h, w_ref[...]) + b_ref[...]


def _tc_layer3(p0, p1, deg, hw, w, b):
    return pl.pallas_call(
        _layer3_body,
        grid=(NBLK,),
        in_specs=[
            pl.BlockSpec((BLK, D), lambda i: (i, 0)),
            pl.BlockSpec((BLK, D), lambda i: (i, 0)),
            pl.BlockSpec((BLK, 1), lambda i: (i, 0)),
            pl.BlockSpec((BLK, D), lambda i: (i, 0)),
            pl.BlockSpec((D, D), lambda i: (0, 0)),
            pl.BlockSpec((1, D), lambda i: (0, 0)),
        ],
        out_specs=pl.BlockSpec((BLK, D), lambda i: (i, 0)),
        out_shape=jax.ShapeDtypeStruct((NP, D), jnp.float32),
    )(p0, p1, deg, hw, w, b)


def _passa_body(p0, p1, deg, hw, aw, ab, batch, o_h, o_sc, o_m):
    i = pl.program_id(0)

    @pl.when(i == 0)
    def _():
        o_m[...] = jnp.full((8, G), NEG, jnp.float32)

    h = jax.nn.relu((p0[...] + p1[...]) / deg[...] + hw[...])
    o_h[...] = h
    sc = _dot(h, aw[...]) + ab[...]
    o_sc[...] = sc
    onehot = batch[...] == lax.broadcasted_iota(jnp.int32, (1, G), 1)
    masked = jnp.where(onehot, sc, NEG)
    mblk = jnp.max(masked, axis=0, keepdims=True)
    o_m[...] = jnp.maximum(o_m[...], jnp.broadcast_to(mblk, (8, G)))


def _tc_passa(p0, p1, deg, hw, aw, ab, batch2d):
    return pl.pallas_call(
        _passa_body,
        grid=(NBLK,),
        in_specs=[
            pl.BlockSpec((BLK, D), lambda i: (i, 0)),
            pl.BlockSpec((BLK, D), lambda i: (i, 0)),
            pl.BlockSpec((BLK, 1), lambda i: (i, 0)),
            pl.BlockSpec((BLK, D), lambda i: (i, 0)),
            pl.BlockSpec((D, 1), lambda i: (0, 0)),
            pl.BlockSpec((1, 1), lambda i: (0, 0)),
            pl.BlockSpec((BLK, 1), lambda i: (i, 0)),
        ],
        out_specs=[
            pl.BlockSpec((BLK, D), lambda i: (i, 0)),
            pl.BlockSpec((BLK, 1), lambda i: (i, 0)),
            pl.BlockSpec((8, G), lambda i: (0, 0)),
        ],
        out_shape=[
            jax.ShapeDtypeStruct((NP, D), jnp.float32),
            jax.ShapeDtypeStruct((NP, 1), jnp.float32),
            jax.ShapeDtypeStruct((8, G), jnp.float32),
        ],
    )(p0, p1, deg, hw, aw, ab, batch2d)


def _passb_body(h, sc, batch, batcht, mcol,
                ndw1, ndb1, ndw2, ndb2, new1, neb1, new2, neb2,
                o_node, o_graph, z_acc, s_acc):
    i = pl.program_id(0)

    @pl.when(i == 0)
    def _():
        z_acc[...] = jnp.zeros((G, 1), jnp.float32)
        s_acc[...] = jnp.zeros((G, D), jnp.float32)
        o_graph[...] = jnp.zeros((G, 1), jnp.float32)

    hb = h[...]
    bcol = batch[...]
    onehot = (bcol == lax.broadcasted_iota(jnp.int32, (1, G), 1)
              ).astype(jnp.float32)
    onehot_t = (lax.broadcasted_iota(jnp.int32, (G, 1), 0) == batcht[...]
                ).astype(jnp.float32)
    mnode = _dotx(onehot, mcol[...])
    valid = bcol < G
    ex = jnp.where(valid, jnp.exp(sc[...] - mnode), 0.0)
    z_acc[...] += _dotx(onehot_t, ex)
    s_acc[...] += _dotx(onehot_t, hb * ex)
    nhid = jax.nn.relu(
        _dot(hb, ndw1[...]) + ndb1[...])
    o_node[...] = (
        _dot(nhid, ndw2[...])
        + ndb2[...])

    @pl.when(i == pl.num_programs(0) - 1)
    def _():
        z = z_acc[...]
        ge = jnp.where(z > 0.0, s_acc[...] / z, 0.0)
        ghid = jax.nn.relu(
            _dot(ge, new1[...])
            + neb1[...])
        o_graph[...] = (
            _dot(ghid, new2[...])
            + neb2[...])


def _tc_passb(h, sc, batch2d, batcht, mcol,
              ndw1, ndb1, ndw2, ndb2, new1, neb1, new2, neb2):
    return pl.pallas_call(
        _passb_body,
        grid=(NBLK,),
        in_specs=[
            pl.BlockSpec((BLK, D), lambda i: (i, 0)),
            pl.BlockSpec((BLK, 1), lambda i: (i, 0)),
            pl.BlockSpec((BLK, 1), lambda i: (i, 0)),
            pl.BlockSpec((1, BLK), lambda i: (0, i)),
            pl.BlockSpec((G, 1), lambda i: (0, 0)),
            pl.BlockSpec((D, G), lambda i: (0, 0)),
            pl.BlockSpec((1, G), lambda i: (0, 0)),
            pl.BlockSpec((G, 1), lambda i: (0, 0)),
            pl.BlockSpec((1, 1), lambda i: (0, 0)),
            pl.BlockSpec((D, G), lambda i: (0, 0)),
            pl.BlockSpec((1, G), lambda i: (0, 0)),
            pl.BlockSpec((G, 1), lambda i: (0, 0)),
            pl.BlockSpec((1, 1), lambda i: (0, 0)),
        ],
        out_specs=[
            pl.BlockSpec((BLK, 1), lambda i: (i, 0)),
            pl.BlockSpec((G, 1), lambda i: (0, 0)),
        ],
        out_shape=[
            jax.ShapeDtypeStruct((NP, 1), jnp.float32),
            jax.ShapeDtypeStruct((G, 1), jnp.float32),
        ],
        scratch_shapes=[
            pltpu.VMEM((G, 1), jnp.float32),
            pltpu.VMEM((G, D), jnp.float32),
        ],
    )(h, sc, batch2d, batcht, mcol,
      ndw1, ndb1, ndw2, ndb2, new1, neb1, new2, neb2)


def kernel(x, edge_index, batch, W0, b0, W1, b1, W2, b2, attn_w, attn_b,
           ne_w1, ne_b1, ne_w2, ne_b2, nd_w1, nd_b1, nd_w2, nd_b2):
    f32 = jnp.float32
    xp = jnp.pad(x, ((0, NP - N), (0, 0)))
    pad_e = EP - E
    src = jnp.concatenate([edge_index[0], jnp.zeros((pad_e,), jnp.int32)])
    pad_dst = N + jnp.arange(pad_e, dtype=jnp.int32) % (NP - N)
    dst = jnp.concatenate([edge_index[1], pad_dst])
    batch_p = jnp.pad(batch, (0, NP - N), constant_values=G)
    batch2d = batch_p.reshape(NP, 1)
    batcht = batch_p.reshape(1, NP)
    z2d = jnp.zeros((ROWS_PER_TILE, D), f32)
    z1d = jnp.zeros((ROWS_PER_TILE,), f32)

    b0r = b0.reshape(1, D)
    b1r = b1.reshape(1, D)
    b2r = b2.reshape(1, D)
    abr = attn_b.reshape(1, 1)
    neb1r = ne_b1.reshape(1, G)
    neb2r = ne_b2.reshape(1, 1)
    ndb1r = nd_b1.reshape(1, G)
    ndb2r = nd_b2.reshape(1, 1)

    hw0 = _tc_mm_bias(xp, W0, b0r)

    part, degpart = _sc_agg_deg(hw0, src, dst, z2d, z1d)
    p0, p1 = part[:NP], part[NP:]
    d0 = degpart[:NP].reshape(NP, 1)
    d1 = degpart[NP:].reshape(NP, 1)

    hw1, deg = _tc_layer2(p0, p1, d0, d1, hw0, W1, b1r)

    part = _sc_agg(hw1, src, dst, z2d)
    hw2 = _tc_layer3(part[:NP], part[NP:], deg, hw1, W2, b2r)

    part = _sc_agg(hw2, src, dst, z2d)
    h3, scores, m = _tc_passa(part[:NP], part[NP:], deg, hw2,
                              attn_w, abr, batch2d)
    mcol = m[0:1].reshape(G, 1)

    node_logits, graph_logits = _tc_passb(
        h3, scores, batch2d, batcht, mcol,
        nd_w1, ndb1r, nd_w2, ndb2r, ne_w1, neb1r, ne_w2, neb2r)

    return (graph_logits, node_logits[:N])


# speedup vs baseline: 1.1685x; 1.1685x over previous
import functools

import jax
import jax.numpy as jnp
from jax import lax
from jax.experimental import pallas as pl
from jax.experimental.pallas import tpu as pltpu
from jax.experimental.pallas import tpu_sc as plsc

N = 10000
NP = 10240
D = 128
G = 64
E = 320000
NC = 2
NS = 16
NW = NC * NS
CH = 128
K0 = 111
K1 = 47
SRCB_W = max(K0, K1) * CH
EP = NS * (K0 + K1) * CH
ROWS_PER_TILE = NP // NS
BLK = 256
NBLK = NP // BLK
NEG = -3.4e38


def _sc_mesh():
    return plsc.VectorSubcoreMesh(
        core_axis_name="c", subcore_axis_name="s", num_cores=NC)


@functools.lru_cache(maxsize=None)
def _build_sc_agg(with_deg):
    out_type = [jax.ShapeDtypeStruct((2 * NP, D), jnp.float32)]
    scratch = [
        pltpu.VMEM((SRCB_W,), jnp.int32),
        pltpu.VMEM((CH,), jnp.int32),
        pltpu.VMEM((CH,), jnp.int32),
        pltpu.VMEM((CH, D), jnp.float32),
        pltpu.VMEM((CH, D), jnp.float32),
        pltpu.VMEM_SHARED((NP, D), jnp.float32),
        pltpu.SemaphoreType.DMA,
        pltpu.SemaphoreType.DMA,
        pltpu.SemaphoreType.DMA,
        pltpu.SemaphoreType.DMA,
        pltpu.SemaphoreType.DMA,
        pltpu.SemaphoreType.DMA,
    ]
    if with_deg:
        out_type.append(jax.ShapeDtypeStruct((2 * NP,), jnp.float32))
        scratch += [
            pltpu.VMEM((CH,), jnp.float32),
            pltpu.VMEM_SHARED((NP,), jnp.float32),
            pltpu.SemaphoreType.DMA,
        ]
    body = _sc_agg_deg_body if with_deg else _sc_agg_body
    return functools.partial(
        pl.kernel,
        mesh=_sc_mesh(),
        out_type=tuple(out_type),
        scratch_types=tuple(scratch),
    )(body)


def _sc_agg_deg(hw, src3, dst3, z2d, z1d):
    part, degpart = _build_sc_agg(True)(hw, src3, dst3, z2d, z1d)
    return part, degpart


def _sc_agg(hw, src3, dst3, z2d):
    out = _build_sc_agg(False)(hw, src3, dst3, z2d)
    return out[0] if isinstance(out, (tuple, list)) else out


def _sc_pipeline(nchunk, hw, acc, srcb, dstw, dbase, rows0, rows1,
                 dstv0, dstv1, gs0, gs1, ds0, ds1, deg_issue):
    def srcb_row(j):
        return srcb.at[pl.ds(pl.multiple_of(j * CH, 8), CH)]

    def dst_row(j):
        return dstw.at[pl.ds(pl.multiple_of(dbase + j * CH, 8), CH)]

    def issue(j, rbuf, gsem, dbuf, dsem):
        pltpu.async_copy(hw.at[srcb_row(j)], rbuf, gsem)
        pltpu.async_copy(dst_row(j), dbuf, dsem)

    def wait_in(j, rbuf, gsem, dbuf, dsem):
        pltpu.make_async_copy(hw.at[srcb_row(j)], rbuf, gsem).wait()
        pltpu.make_async_copy(dst_row(j), dbuf, dsem).wait()

    def scatter(rbuf, dbuf):
        pltpu.sync_copy(rbuf, acc.at[dbuf], add=True)
        deg_issue(dbuf)

    issue(0, rows0, gs0, dstv0, ds0)

    def body(i, carry):
        j0 = i * 2
        issue(j0 + 1, rows1, gs1, dstv1, ds1)
        wait_in(j0, rows0, gs0, dstv0, ds0)
        scatter(rows0, dstv0)
        issue(j0 + 2, rows0, gs0, dstv0, ds0)
        wait_in(j0 + 1, rows1, gs1, dstv1, ds1)
        scatter(rows1, dstv1)
        return carry

    lax.fori_loop(0, (nchunk - 1) // 2, body, 0)
    assert nchunk % 2 == 1
    wait_in(nchunk - 1, rows0, gs0, dstv0, ds0)
    scatter(rows0, dstv0)


def _sc_common(hw, src3, dst3, z2d, part, srcb, dstv0, dstv1, rows0, rows1,
               acc, gs0, gs1, ds0, ds1, ss0, ss1, deg_issue, deg_drain):
    c = lax.axis_index("c")
    s = lax.axis_index("s")
    pltpu.sync_copy(z2d, acc.at[pl.ds(s * ROWS_PER_TILE, ROWS_PER_TILE)])
    plsc.subcore_barrier()
    base0 = pl.multiple_of(s * (K0 * CH), 8)
    base1 = pl.multiple_of(NS * (K0 * CH) + s * (K1 * CH), 8)

    @pl.when(c == 0)
    def _():
        pltpu.sync_copy(src3.at[pl.ds(base0, K0 * CH)],
                        srcb.at[pl.ds(0, K0 * CH)])
        _sc_pipeline(K0, hw, acc, srcb, dst3, base0, rows0, rows1,
                     dstv0, dstv1, gs0, gs1, ds0, ds1, deg_issue)

    @pl.when(c == 1)
    def _():
        pltpu.sync_copy(src3.at[pl.ds(base1, K1 * CH)],
                        srcb.at[pl.ds(0, K1 * CH)])
        _sc_pipeline(K1, hw, acc, srcb, dst3, base1, rows0, rows1,
                     dstv0, dstv1, gs0, gs1, ds0, ds1, deg_issue)

    plsc.subcore_barrier()
    ro = pl.multiple_of(s * ROWS_PER_TILE, 8)
    oo = pl.multiple_of(c * NP + s * ROWS_PER_TILE, 8)
    pltpu.sync_copy(acc.at[pl.ds(ro, ROWS_PER_TILE)],
                    part.at[pl.ds(oo, ROWS_PER_TILE)])
    return c, s, ro, oo


def _sc_agg_deg_body(hw, src3, dst3, z2d, z1d, part, degpart,
                     srcb, dstv0, dstv1, rows0, rows1, acc,
                     gs0, gs1, ds0, ds1, ss0, ss1, ones_v, dacc, dg):
    s = lax.axis_index("s")
    pltpu.sync_copy(z1d, dacc.at[pl.ds(s * ROWS_PER_TILE, ROWS_PER_TILE)])
    for t in range(CH // 16):
        ones_v[pl.ds(t * 16, 16)] = jnp.full((16,), 1.0, jnp.float32)

    def deg_issue(dbuf):
        pltpu.sync_copy(ones_v, dacc.at[dbuf], add=True)

    def deg_drain():
        pass

    _, _, ro, oo = _sc_common(hw, src3, dst3, z2d, part, srcb, dstv0, dstv1,
                              rows0, rows1, acc, gs0, gs1, ds0, ds1,
                              ss0, ss1, deg_issue, deg_drain)
    pltpu.sync_copy(dacc.at[pl.ds(ro, ROWS_PER_TILE)],
                    degpart.at[pl.ds(oo, ROWS_PER_TILE)])


def _sc_agg_body(hw, src3, dst3, z2d, part, srcb, dstv0, dstv1, rows0, rows1,
                 acc, gs0, gs1, ds0, ds1, ss0, ss1):
    _sc_common(hw, src3, dst3, z2d, part, srcb, dstv0, dstv1, rows0, rows1,
               acc, gs0, gs1, ds0, ds1, ss0, ss1,
               lambda dbuf: None, lambda: None)


def _dot(a, b):
    return jnp.dot(a, b, preferred_element_type=jnp.float32)


def _dotx(a, b):
    return jnp.dot(a, b, preferred_element_type=jnp.float32,
                   precision=lax.Precision.HIGHEST)


def _mm_bias_body(x_ref, w_ref, b_ref, o_ref):
    o_ref[...] = (
        _dot(x_ref[...], w_ref[...])
        + b_ref[...]
    )


def _tc_mm_bias(x, w, b):
    return pl.pallas_call(
        _mm_bias_body,
        grid=(NBLK,),
        in_specs=[
            pl.BlockSpec((BLK, D), lambda i: (i, 0)),
            pl.BlockSpec((D, D), lambda i: (0, 0)),
            pl.BlockSpec((1, D), lambda i: (0, 0)),
        ],
        out_specs=pl.BlockSpec((BLK, D), lambda i: (i, 0)),
        out_shape=jax.ShapeDtypeStruct((NP, D), jnp.float32),
    )(x, w, b)


def _layer2_body(p0, p1, d0, d1, hw, w_ref, b_ref, o_hw, o_deg):
    deg = jnp.maximum(d0[...] + d1[...], 1.0)
    h = jax.nn.relu((p0[...] + p1[...]) / deg + hw[...])
    o_hw[...] = _dot(h, w_ref[...]) + b_ref[...]
    o_deg[...] = deg


def _tc_layer2(p0, p1, d0, d1, hw, w, b):
    return pl.pallas_call(
        _layer2_body,
        grid=(NBLK,),
        in_specs=[
            pl.BlockSpec((BLK, D), lambda i: (i, 0)),
            pl.BlockSpec((BLK, D), lambda i: (i, 0)),
            pl.BlockSpec((BLK, 1), lambda i: (i, 0)),
            pl.BlockSpec((BLK, 1), lambda i: (i, 0)),
            pl.BlockSpec((BLK, D), lambda i: (i, 0)),
            pl.BlockSpec((D, D), lambda i: (0, 0)),
            pl.BlockSpec((1, D), lambda i: (0, 0)),
        ],
        out_specs=[
            pl.BlockSpec((BLK, D), lambda i: (i, 0)),
            pl.BlockSpec((BLK, 1), lambda i: (i, 0)),
        ],
        out_shape=[
            jax.ShapeDtypeStruct((NP, D), jnp.float32),
            jax.ShapeDtypeStruct((NP, 1), jnp.float32),
        ],
    )(p0, p1, d0, d1, hw, w, b)


def _layer3_body(p0, p1, deg, hw, w_ref, b_ref, o_hw):
    h = jax.nn.relu((p0[...] + p1[...]) / deg[...] + hw[...])
    o_hw[...] = _dot(h, w_ref[...]) + b_ref[...]


def _tc_layer3(p0, p1, deg, hw, w, b):
    return pl.pallas_call(
        _layer3_body,
        grid=(NBLK,),
        in_specs=[
            pl.BlockSpec((BLK, D), lambda i: (i, 0)),
            pl.BlockSpec((BLK, D), lambda i: (i, 0)),
            pl.BlockSpec((BLK, 1), lambda i: (i, 0)),
            pl.BlockSpec((BLK, D), lambda i: (i, 0)),
            pl.BlockSpec((D, D), lambda i: (0, 0)),
            pl.BlockSpec((1, D), lambda i: (0, 0)),
        ],
        out_specs=pl.BlockSpec((BLK, D), lambda i: (i, 0)),
        out_shape=jax.ShapeDtypeStruct((NP, D), jnp.float32),
    )(p0, p1, deg, hw, w, b)


def _passa_body(p0, p1, deg, hw, aw, ab, batch, o_h, o_sc, o_m):
    i = pl.program_id(0)

    @pl.when(i == 0)
    def _():
        o_m[...] = jnp.full((8, G), NEG, jnp.float32)

    h = jax.nn.relu((p0[...] + p1[...]) / deg[...] + hw[...])
    o_h[...] = h
    sc = _dot(h, aw[...]) + ab[...]
    o_sc[...] = sc
    onehot = batch[...] == lax.broadcasted_iota(jnp.int32, (1, G), 1)
    masked = jnp.where(onehot, sc, NEG)
    mblk = jnp.max(masked, axis=0, keepdims=True)
    o_m[...] = jnp.maximum(o_m[...], jnp.broadcast_to(mblk, (8, G)))


def _tc_passa(p0, p1, deg, hw, aw, ab, batch2d):
    return pl.pallas_call(
        _passa_body,
        grid=(NBLK,),
        in_specs=[
            pl.BlockSpec((BLK, D), lambda i: (i, 0)),
            pl.BlockSpec((BLK, D), lambda i: (i, 0)),
            pl.BlockSpec((BLK, 1), lambda i: (i, 0)),
            pl.BlockSpec((BLK, D), lambda i: (i, 0)),
            pl.BlockSpec((D, 1), lambda i: (0, 0)),
            pl.BlockSpec((1, 1), lambda i: (0, 0)),
            pl.BlockSpec((BLK, 1), lambda i: (i, 0)),
        ],
        out_specs=[
            pl.BlockSpec((BLK, D), lambda i: (i, 0)),
            pl.BlockSpec((BLK, 1), lambda i: (i, 0)),
            pl.BlockSpec((8, G), lambda i: (0, 0)),
        ],
        out_shape=[
            jax.ShapeDtypeStruct((NP, D), jnp.float32),
            jax.ShapeDtypeStruct((NP, 1), jnp.float32),
            jax.ShapeDtypeStruct((8, G), jnp.float32),
        ],
    )(p0, p1, deg, hw, aw, ab, batch2d)


def _passb_body(h, sc, batch, batcht, mcol,
                ndw1, ndb1, ndw2, ndb2, new1, neb1, new2, neb2,
                o_node, o_graph, z_acc, s_acc):
    i = pl.program_id(0)

    @pl.when(i == 0)
    def _():
        z_acc[...] = jnp.zeros((G, 1), jnp.float32)
        s_acc[...] = jnp.zeros((G, D), jnp.float32)
        o_graph[...] = jnp.zeros((G, 1), jnp.float32)

    hb = h[...]
    bcol = batch[...]
    onehot = (bcol == lax.broadcasted_iota(jnp.int32, (1, G), 1)
              ).astype(jnp.float32)
    onehot_t = (lax.broadcasted_iota(jnp.int32, (G, 1), 0) == batcht[...]
                ).astype(jnp.float32)
    mnode = _dotx(onehot, mcol[...])
    valid = bcol < G
    ex = jnp.where(valid, jnp.exp(sc[...] - mnode), 0.0)
    z_acc[...] += _dotx(onehot_t, ex)
    s_acc[...] += _dotx(onehot_t, hb * ex)
    nhid = jax.nn.relu(
        _dot(hb, ndw1[...]) + ndb1[...])
    o_node[...] = (
        _dot(nhid, ndw2[...])
        + ndb2[...])

    @pl.when(i == pl.num_programs(0) - 1)
    def _():
        z = z_acc[...]
        ge = jnp.where(z > 0.0, s_acc[...] / z, 0.0)
        ghid = jax.nn.relu(
            _dot(ge, new1[...])
            + neb1[...])
        o_graph[...] = (
            _dot(ghid, new2[...])
            + neb2[...])


def _tc_passb(h, sc, batch2d, batcht, mcol,
              ndw1, ndb1, ndw2, ndb2, new1, neb1, new2, neb2):
    return pl.pallas_call(
        _passb_body,
        grid=(NBLK,),
        in_specs=[
            pl.BlockSpec((BLK, D), lambda i: (i, 0)),
            pl.BlockSpec((BLK, 1), lambda i: (i, 0)),
            pl.BlockSpec((BLK, 1), lambda i: (i, 0)),
            pl.BlockSpec((1, BLK), lambda i: (0, i)),
            pl.BlockSpec((G, 1), lambda i: (0, 0)),
            pl.BlockSpec((D, G), lambda i: (0, 0)),
            pl.BlockSpec((1, G), lambda i: (0, 0)),
            pl.BlockSpec((G, 1), lambda i: (0, 0)),
            pl.BlockSpec((1, 1), lambda i: (0, 0)),
            pl.BlockSpec((D, G), lambda i: (0, 0)),
            pl.BlockSpec((1, G), lambda i: (0, 0)),
            pl.BlockSpec((G, 1), lambda i: (0, 0)),
            pl.BlockSpec((1, 1), lambda i: (0, 0)),
        ],
        out_specs=[
            pl.BlockSpec((BLK, 1), lambda i: (i, 0)),
            pl.BlockSpec((G, 1), lambda i: (0, 0)),
        ],
        out_shape=[
            jax.ShapeDtypeStruct((NP, 1), jnp.float32),
            jax.ShapeDtypeStruct((G, 1), jnp.float32),
        ],
        scratch_shapes=[
            pltpu.VMEM((G, 1), jnp.float32),
            pltpu.VMEM((G, D), jnp.float32),
        ],
    )(h, sc, batch2d, batcht, mcol,
      ndw1, ndb1, ndw2, ndb2, new1, neb1, new2, neb2)


def kernel(x, edge_index, batch, W0, b0, W1, b1, W2, b2, attn_w, attn_b,
           ne_w1, ne_b1, ne_w2, ne_b2, nd_w1, nd_b1, nd_w2, nd_b2):
    f32 = jnp.float32
    xp = jnp.pad(x, ((0, NP - N), (0, 0)))
    pad_e = EP - E
    src = jnp.concatenate([edge_index[0], jnp.zeros((pad_e,), jnp.int32)])
    pad_dst = N + jnp.arange(pad_e, dtype=jnp.int32) % (NP - N)
    dst = jnp.concatenate([edge_index[1], pad_dst])
    batch_p = jnp.pad(batch, (0, NP - N), constant_values=G)
    batch2d = batch_p.reshape(NP, 1)
    batcht = batch_p.reshape(1, NP)
    z2d = jnp.zeros((ROWS_PER_TILE, D), f32)
    z1d = jnp.zeros((ROWS_PER_TILE,), f32)

    b0r = b0.reshape(1, D)
    b1r = b1.reshape(1, D)
    b2r = b2.reshape(1, D)
    abr = attn_b.reshape(1, 1)
    neb1r = ne_b1.reshape(1, G)
    neb2r = ne_b2.reshape(1, 1)
    ndb1r = nd_b1.reshape(1, G)
    ndb2r = nd_b2.reshape(1, 1)

    hw0 = _tc_mm_bias(xp, W0, b0r)

    part, degpart = _sc_agg_deg(hw0, src, dst, z2d, z1d)
    p0, p1 = part[:NP], part[NP:]
    d0 = degpart[:NP].reshape(NP, 1)
    d1 = degpart[NP:].reshape(NP, 1)

    hw1, deg = _tc_layer2(p0, p1, d0, d1, hw0, W1, b1r)

    part = _sc_agg(hw1, src, dst, z2d)
    hw2 = _tc_layer3(part[:NP], part[NP:], deg, hw1, W2, b2r)

    part = _sc_agg(hw2, src, dst, z2d)
    h3, scores, m = _tc_passa(part[:NP], part[NP:], deg, hw2,
                              attn_w, abr, batch2d)
    mcol = m[0:1].reshape(G, 1)

    node_logits, graph_logits = _tc_passb(
        h3, scores, batch2d, batcht, mcol,
        nd_w1, ndb1r, nd_w2, ndb2r, ne_w1, neb1r, ne_w2, neb2r)

    return (graph_logits, node_logits[:N])


# speedup vs baseline: 1.2002x; 1.0271x over previous
import functools

import jax
import jax.numpy as jnp
from jax import lax
from jax.experimental import pallas as pl
from jax.experimental.pallas import tpu as pltpu
from jax.experimental.pallas import tpu_sc as plsc

N = 10000
NP = 10240
D = 128
G = 64
E = 320000
NC = 2
NS = 16
NW = NC * NS
CH = 128
K0 = 119
K1 = 39
SRCB_W = max(K0, K1) * CH
EP = NS * (K0 + K1) * CH
ROWS_PER_TILE = NP // NS
BLK = 256
NBLK = NP // BLK
NEG = -3.4e38


def _sc_mesh():
    return plsc.VectorSubcoreMesh(
        core_axis_name="c", subcore_axis_name="s", num_cores=NC)


@functools.lru_cache(maxsize=None)
def _build_sc_agg(with_deg):
    out_type = [jax.ShapeDtypeStruct((2 * NP, D), jnp.float32)]
    scratch = [
        pltpu.VMEM((SRCB_W,), jnp.int32),
        pltpu.VMEM((CH,), jnp.int32),
        pltpu.VMEM((CH,), jnp.int32),
        pltpu.VMEM((CH, D), jnp.float32),
        pltpu.VMEM((CH, D), jnp.float32),
        pltpu.VMEM_SHARED((NP, D), jnp.float32),
        pltpu.SemaphoreType.DMA,
        pltpu.SemaphoreType.DMA,
        pltpu.SemaphoreType.DMA,
        pltpu.SemaphoreType.DMA,
        pltpu.SemaphoreType.DMA,
        pltpu.SemaphoreType.DMA,
    ]
    if with_deg:
        out_type.append(jax.ShapeDtypeStruct((2 * NP,), jnp.float32))
        scratch += [
            pltpu.VMEM((CH,), jnp.float32),
            pltpu.VMEM_SHARED((NP,), jnp.float32),
            pltpu.SemaphoreType.DMA,
        ]
    body = _sc_agg_deg_body if with_deg else _sc_agg_body
    return functools.partial(
        pl.kernel,
        mesh=_sc_mesh(),
        out_type=tuple(out_type),
        scratch_types=tuple(scratch),
    )(body)


def _sc_agg_deg(hw, src3, dst3, z2d, z1d):
    part, degpart = _build_sc_agg(True)(hw, src3, dst3, z2d, z1d)
    return part, degpart


def _sc_agg(hw, src3, dst3, z2d):
    out = _build_sc_agg(False)(hw, src3, dst3, z2d)
    return out[0] if isinstance(out, (tuple, list)) else out


def _sc_pipeline(nchunk, hw, acc, srcb, dstw, dbase, rows0, rows1,
                 dstv0, dstv1, gs0, gs1, ds0, ds1, deg_issue):
    def srcb_row(j):
        return srcb.at[pl.ds(pl.multiple_of(j * CH, 8), CH)]

    def dst_row(j):
        return dstw.at[pl.ds(pl.multiple_of(dbase + j * CH, 8), CH)]

    def issue(j, rbuf, gsem, dbuf, dsem):
        pltpu.async_copy(hw.at[srcb_row(j)], rbuf, gsem)
        pltpu.async_copy(dst_row(j), dbuf, dsem)

    def wait_in(j, rbuf, gsem, dbuf, dsem):
        pltpu.make_async_copy(hw.at[srcb_row(j)], rbuf, gsem).wait()
        pltpu.make_async_copy(dst_row(j), dbuf, dsem).wait()

    def scatter(rbuf, dbuf):
        pltpu.sync_copy(rbuf, acc.at[dbuf], add=True)
        deg_issue(dbuf)

    issue(0, rows0, gs0, dstv0, ds0)

    def body(i, carry):
        j0 = i * 2
        issue(j0 + 1, rows1, gs1, dstv1, ds1)
        wait_in(j0, rows0, gs0, dstv0, ds0)
        scatter(rows0, dstv0)
        issue(j0 + 2, rows0, gs0, dstv0, ds0)
        wait_in(j0 + 1, rows1, gs1, dstv1, ds1)
        scatter(rows1, dstv1)
        return carry

    lax.fori_loop(0, (nchunk - 1) // 2, body, 0)
    assert nchunk % 2 == 1
    wait_in(nchunk - 1, rows0, gs0, dstv0, ds0)
    scatter(rows0, dstv0)


def _sc_common(hw, src3, dst3, z2d, part, srcb, dstv0, dstv1, rows0, rows1,
               acc, gs0, gs1, ds0, ds1, ss0, ss1, deg_issue, deg_drain):
    c = lax.axis_index("c")
    s = lax.axis_index("s")
    pltpu.sync_copy(z2d, acc.at[pl.ds(s * ROWS_PER_TILE, ROWS_PER_TILE)])
    plsc.subcore_barrier()
    base0 = pl.multiple_of(s * (K0 * CH), 8)
    base1 = pl.multiple_of(NS * (K0 * CH) + s * (K1 * CH), 8)

    @pl.when(c == 0)
    def _():
        pltpu.sync_copy(src3.at[pl.ds(base0, K0 * CH)],
                        srcb.at[pl.ds(0, K0 * CH)])
        _sc_pipeline(K0, hw, acc, srcb, dst3, base0, rows0, rows1,
                     dstv0, dstv1, gs0, gs1, ds0, ds1, deg_issue)

    @pl.when(c == 1)
    def _():
        pltpu.sync_copy(src3.at[pl.ds(base1, K1 * CH)],
                        srcb.at[pl.ds(0, K1 * CH)])
        _sc_pipeline(K1, hw, acc, srcb, dst3, base1, rows0, rows1,
                     dstv0, dstv1, gs0, gs1, ds0, ds1, deg_issue)

    plsc.subcore_barrier()
    ro = pl.multiple_of(s * ROWS_PER_TILE, 8)
    oo = pl.multiple_of(c * NP + s * ROWS_PER_TILE, 8)
    pltpu.sync_copy(acc.at[pl.ds(ro, ROWS_PER_TILE)],
                    part.at[pl.ds(oo, ROWS_PER_TILE)])
    return c, s, ro, oo


def _sc_agg_deg_body(hw, src3, dst3, z2d, z1d, part, degpart,
                     srcb, dstv0, dstv1, rows0, rows1, acc,
                     gs0, gs1, ds0, ds1, ss0, ss1, ones_v, dacc, dg):
    s = lax.axis_index("s")
    pltpu.sync_copy(z1d, dacc.at[pl.ds(s * ROWS_PER_TILE, ROWS_PER_TILE)])
    for t in range(CH // 16):
        ones_v[pl.ds(t * 16, 16)] = jnp.full((16,), 1.0, jnp.float32)

    def deg_issue(dbuf):
        pltpu.sync_copy(ones_v, dacc.at[dbuf], add=True)

    def deg_drain():
        pass

    _, _, ro, oo = _sc_common(hw, src3, dst3, z2d, part, srcb, dstv0, dstv1,
                              rows0, rows1, acc, gs0, gs1, ds0, ds1,
                              ss0, ss1, deg_issue, deg_drain)
    pltpu.sync_copy(dacc.at[pl.ds(ro, ROWS_PER_TILE)],
                    degpart.at[pl.ds(oo, ROWS_PER_TILE)])


def _sc_agg_body(hw, src3, dst3, z2d, part, srcb, dstv0, dstv1, rows0, rows1,
                 acc, gs0, gs1, ds0, ds1, ss0, ss1):
    _sc_common(hw, src3, dst3, z2d, part, srcb, dstv0, dstv1, rows0, rows1,
               acc, gs0, gs1, ds0, ds1, ss0, ss1,
               lambda dbuf: None, lambda: None)


def _dot(a, b):
    return jnp.dot(a, b, preferred_element_type=jnp.float32)


def _dotx(a, b):
    return jnp.dot(a, b, preferred_element_type=jnp.float32,
                   precision=lax.Precision.HIGHEST)


def _mm_bias_body(x_ref, w_ref, b_ref, o_ref):
    o_ref[...] = (
        _dot(x_ref[...], w_ref[...])
        + b_ref[...]
    )


def _tc_mm_bias(x, w, b):
    return pl.pallas_call(
        _mm_bias_body,
        grid=(NBLK,),
        in_specs=[
            pl.BlockSpec((BLK, D), lambda i: (i, 0)),
            pl.BlockSpec((D, D), lambda i: (0, 0)),
            pl.BlockSpec((1, D), lambda i: (0, 0)),
        ],
        out_specs=pl.BlockSpec((BLK, D), lambda i: (i, 0)),
        out_shape=jax.ShapeDtypeStruct((NP, D), jnp.float32),
    )(x, w, b)


def _layer2_body(p0, p1, d0, d1, hw, w_ref, b_ref, o_hw, o_deg):
    deg = jnp.maximum(d0[...] + d1[...], 1.0)
    h = jax.nn.relu((p0[...] + p1[...]) / deg + hw[...])
    o_hw[...] = _dot(h, w_ref[...]) + b_ref[...]
    o_deg[...] = deg


def _tc_layer2(p0, p1, d0, d1, hw, w, b):
    return pl.pallas_call(
        _layer2_body,
        grid=(NBLK,),
        in_specs=[
            pl.BlockSpec((BLK, D), lambda i: (i, 0)),
            pl.BlockSpec((BLK, D), lambda i: (i, 0)),
            pl.BlockSpec((BLK, 1), lambda i: (i, 0)),
            pl.BlockSpec((BLK, 1), lambda i: (i, 0)),
            pl.BlockSpec((BLK, D), lambda i: (i, 0)),
            pl.BlockSpec((D, D), lambda i: (0, 0)),
            pl.BlockSpec((1, D), lambda i: (0, 0)),
        ],
        out_specs=[
            pl.BlockSpec((BLK, D), lambda i: (i, 0)),
            pl.BlockSpec((BLK, 1), lambda i: (i, 0)),
        ],
        out_shape=[
            jax.ShapeDtypeStruct((NP, D), jnp.float32),
            jax.ShapeDtypeStruct((NP, 1), jnp.float32),
        ],
    )(p0, p1, d0, d1, hw, w, b)


def _layer3_body(p0, p1, deg, hw, w_ref, b_ref, o_hw):
    h = jax.nn.relu((p0[...] + p1[...]) / deg[...] + hw[...])
    o_hw[...] = _dot(h, w_ref[...]) + b_ref[...]


def _tc_layer3(p0, p1, deg, hw, w, b):
    return pl.pallas_call(
        _layer3_body,
        grid=(NBLK,),
        in_specs=[
            pl.BlockSpec((BLK, D), lambda i: (i, 0)),
            pl.BlockSpec((BLK, D), lambda i: (i, 0)),
            pl.BlockSpec((BLK, 1), lambda i: (i, 0)),
            pl.BlockSpec((BLK, D), lambda i: (i, 0)),
            pl.BlockSpec((D, D), lambda i: (0, 0)),
            pl.BlockSpec((1, D), lambda i: (0, 0)),
        ],
        out_specs=pl.BlockSpec((BLK, D), lambda i: (i, 0)),
        out_shape=jax.ShapeDtypeStruct((NP, D), jnp.float32),
    )(p0, p1, deg, hw, w, b)


def _passa_body(p0, p1, deg, hw, aw, ab, batch, o_h, o_sc, o_m):
    i = pl.program_id(0)

    @pl.when(i == 0)
    def _():
        o_m[...] = jnp.full((8, G), NEG, jnp.float32)

    h = jax.nn.relu((p0[...] + p1[...]) / deg[...] + hw[...])
    o_h[...] = h
    sc = _dot(h, aw[...]) + ab[...]
    o_sc[...] = sc
    onehot = batch[...] == lax.broadcasted_iota(jnp.int32, (1, G), 1)
    masked = jnp.where(onehot, sc, NEG)
    mblk = jnp.max(masked, axis=0, keepdims=True)
    o_m[...] = jnp.maximum(o_m[...], jnp.broadcast_to(mblk, (8, G)))


def _tc_passa(p0, p1, deg, hw, aw, ab, batch2d):
    return pl.pallas_call(
        _passa_body,
        grid=(NBLK,),
        in_specs=[
            pl.BlockSpec((BLK, D), lambda i: (i, 0)),
            pl.BlockSpec((BLK, D), lambda i: (i, 0)),
            pl.BlockSpec((BLK, 1), lambda i: (i, 0)),
            pl.BlockSpec((BLK, D), lambda i: (i, 0)),
            pl.BlockSpec((D, 1), lambda i: (0, 0)),
            pl.BlockSpec((1, 1), lambda i: (0, 0)),
            pl.BlockSpec((BLK, 1), lambda i: (i, 0)),
        ],
        out_specs=[
            pl.BlockSpec((BLK, D), lambda i: (i, 0)),
            pl.BlockSpec((BLK, 1), lambda i: (i, 0)),
            pl.BlockSpec((8, G), lambda i: (0, 0)),
        ],
        out_shape=[
            jax.ShapeDtypeStruct((NP, D), jnp.float32),
            jax.ShapeDtypeStruct((NP, 1), jnp.float32),
            jax.ShapeDtypeStruct((8, G), jnp.float32),
        ],
    )(p0, p1, deg, hw, aw, ab, batch2d)


def _passb_body(h, sc, batch, batcht, mcol,
                ndw1, ndb1, ndw2, ndb2, new1, neb1, new2, neb2,
                o_node, o_graph, z_acc, s_acc):
    i = pl.program_id(0)

    @pl.when(i == 0)
    def _():
        z_acc[...] = jnp.zeros((G, 1), jnp.float32)
        s_acc[...] = jnp.zeros((G, D), jnp.float32)
        o_graph[...] = jnp.zeros((G, 1), jnp.float32)

    hb = h[...]
    bcol = batch[...]
    onehot = (bcol == lax.broadcasted_iota(jnp.int32, (1, G), 1)
              ).astype(jnp.float32)
    onehot_t = (lax.broadcasted_iota(jnp.int32, (G, 1), 0) == batcht[...]
                ).astype(jnp.float32)
    mnode = _dotx(onehot, mcol[...])
    valid = bcol < G
    ex = jnp.where(valid, jnp.exp(sc[...] - mnode), 0.0)
    z_acc[...] += _dotx(onehot_t, ex)
    s_acc[...] += _dotx(onehot_t, hb * ex)
    nhid = jax.nn.relu(
        _dot(hb, ndw1[...]) + ndb1[...])
    o_node[...] = (
        _dot(nhid, ndw2[...])
        + ndb2[...])

    @pl.when(i == pl.num_programs(0) - 1)
    def _():
        z = z_acc[...]
        ge = jnp.where(z > 0.0, s_acc[...] / z, 0.0)
        ghid = jax.nn.relu(
            _dot(ge, new1[...])
            + neb1[...])
        o_graph[...] = (
            _dot(ghid, new2[...])
            + neb2[...])


def _tc_passb(h, sc, batch2d, batcht, mcol,
              ndw1, ndb1, ndw2, ndb2, new1, neb1, new2, neb2):
    return pl.pallas_call(
        _passb_body,
        grid=(NBLK,),
        in_specs=[
            pl.BlockSpec((BLK, D), lambda i: (i, 0)),
            pl.BlockSpec((BLK, 1), lambda i: (i, 0)),
            pl.BlockSpec((BLK, 1), lambda i: (i, 0)),
            pl.BlockSpec((1, BLK), lambda i: (0, i)),
            pl.BlockSpec((G, 1), lambda i: (0, 0)),
            pl.BlockSpec((D, G), lambda i: (0, 0)),
            pl.BlockSpec((1, G), lambda i: (0, 0)),
            pl.BlockSpec((G, 1), lambda i: (0, 0)),
            pl.BlockSpec((1, 1), lambda i: (0, 0)),
            pl.BlockSpec((D, G), lambda i: (0, 0)),
            pl.BlockSpec((1, G), lambda i: (0, 0)),
            pl.BlockSpec((G, 1), lambda i: (0, 0)),
            pl.BlockSpec((1, 1), lambda i: (0, 0)),
        ],
        out_specs=[
            pl.BlockSpec((BLK, 1), lambda i: (i, 0)),
            pl.BlockSpec((G, 1), lambda i: (0, 0)),
        ],
        out_shape=[
            jax.ShapeDtypeStruct((NP, 1), jnp.float32),
            jax.ShapeDtypeStruct((G, 1), jnp.float32),
        ],
        scratch_shapes=[
            pltpu.VMEM((G, 1), jnp.float32),
            pltpu.VMEM((G, D), jnp.float32),
        ],
    )(h, sc, batch2d, batcht, mcol,
      ndw1, ndb1, ndw2, ndb2, new1, neb1, new2, neb2)


def kernel(x, edge_index, batch, W0, b0, W1, b1, W2, b2, attn_w, attn_b,
           ne_w1, ne_b1, ne_w2, ne_b2, nd_w1, nd_b1, nd_w2, nd_b2):
    f32 = jnp.float32
    xp = jnp.pad(x, ((0, NP - N), (0, 0)))
    pad_e = EP - E
    src = jnp.concatenate([edge_index[0], jnp.zeros((pad_e,), jnp.int32)])
    pad_dst = N + jnp.arange(pad_e, dtype=jnp.int32) % (NP - N)
    dst = jnp.concatenate([edge_index[1], pad_dst])
    batch_p = jnp.pad(batch, (0, NP - N), constant_values=G)
    batch2d = batch_p.reshape(NP, 1)
    batcht = batch_p.reshape(1, NP)
    z2d = jnp.zeros((ROWS_PER_TILE, D), f32)
    z1d = jnp.zeros((ROWS_PER_TILE,), f32)

    b0r = b0.reshape(1, D)
    b1r = b1.reshape(1, D)
    b2r = b2.reshape(1, D)
    abr = attn_b.reshape(1, 1)
    neb1r = ne_b1.reshape(1, G)
    neb2r = ne_b2.reshape(1, 1)
    ndb1r = nd_b1.reshape(1, G)
    ndb2r = nd_b2.reshape(1, 1)

    hw0 = _tc_mm_bias(xp, W0, b0r)

    part, degpart = _sc_agg_deg(hw0, src, dst, z2d, z1d)
    p0, p1 = part[:NP], part[NP:]
    d0 = degpart[:NP].reshape(NP, 1)
    d1 = degpart[NP:].reshape(NP, 1)

    hw1, deg = _tc_layer2(p0, p1, d0, d1, hw0, W1, b1r)

    part = _sc_agg(hw1, src, dst, z2d)
    hw2 = _tc_layer3(part[:NP], part[NP:], deg, hw1, W2, b2r)

    part = _sc_agg(hw2, src, dst, z2d)
    h3, scores, m = _tc_passa(part[:NP], part[NP:], deg, hw2,
                              attn_w, abr, batch2d)
    mcol = m[0:1].reshape(G, 1)

    node_logits, graph_logits = _tc_passb(
        h3, scores, batch2d, batcht, mcol,
        nd_w1, ndb1r, nd_w2, ndb2r, ne_w1, neb1r, ne_w2, neb2r)

    return (graph_logits, node_logits[:N])
